# baseline (speedup 1.0000x reference)
import jax
import jax.numpy as jnp
from jax.experimental import pallas as pl
from jax.experimental.pallas import tpu as pltpu

B = 4
L = 256
R = 256
D_SRC = 128
E_POS = 64
E_CAT = 64
E_SENSE = 64
E_DM = E_POS + E_CAT + E_SENSE
REL = 256
VOCAB = 50
INP = E_DM + D_SRC

A_TILE = 64


def _stage0_kernel(f0, f1, f2, pos, cat, sense, wdm_dep_t, wdm_head_t, dep_b, head_b,
                   dm_emb_out, dm_dep_out, dm_head_out):
    n = f0.shape[0]
    iota = jax.lax.broadcasted_iota(jnp.int32, (n, VOCAB), 1)
    oh0 = (f0[:] == iota).astype(jnp.float32)
    oh1 = (f1[:] == iota).astype(jnp.float32)
    oh2 = (f2[:] == iota).astype(jnp.float32)
    e0 = jnp.dot(oh0, pos[:], preferred_element_type=jnp.float32)
    e1 = jnp.dot(oh1, cat[:], preferred_element_type=jnp.float32)
    e2 = jnp.dot(oh2, sense[:], preferred_element_type=jnp.float32)
    dm = jnp.concatenate([e0, e1, e2], axis=1)
    dm_emb_out[:, :] = dm
    dm_dep_out[:, :] = jnp.dot(dm, wdm_dep_t[:], preferred_element_type=jnp.float32) + dep_b[:]
    dm_head_out[:, :] = jnp.dot(dm, wdm_head_t[:], preferred_element_type=jnp.float32) + head_b[:]


def _dep_kernel(idx_ref, enc_ref, wsrc_t_ref, whsrc_t_ref, dm_dep_ref, dm_head_ref,
                dep_out_ref, head_out_ref):
    a_blk = pl.program_id(1)
    idxc = idx_ref[0]
    iota = jax.lax.broadcasted_iota(jnp.int32, (L, R), 1)
    onehot = (idxc == iota).astype(jnp.bfloat16)
    wsrc = wsrc_t_ref[:]
    whsrc = whsrc_t_ref[:]
    dmdep = dm_dep_ref[0]
    rows = []
    for t in range(A_TILE):
        e = enc_ref[0, t].astype(jnp.bfloat16)
        g = jnp.dot(onehot, e, preferred_element_type=jnp.float32)
        dep_out_ref[t] = (
            jnp.dot(g.astype(jnp.bfloat16), wsrc, preferred_element_type=jnp.float32)
            + dmdep
        )
        ag = a_blk * A_TILE + t
        sel = jax.lax.broadcasted_iota(jnp.int32, (L, D_SRC), 0) == ag
        rows.append(jnp.sum(jnp.where(sel, g, 0.0), axis=0, keepdims=True))
    hs = jnp.concatenate(rows, axis=0).astype(jnp.bfloat16)
    head_out_ref[:, :] = (
        jnp.dot(hs, whsrc, preferred_element_type=jnp.float32) + dm_head_ref[0]
    )


def kernel(feats, index, src_enc, pos_lut, cat_lut, sense_lut, head_W, head_b, dep_W, dep_b):
    f0 = feats[:, 0:1]
    f1 = feats[:, 1:2]
    f2 = feats[:, 2:3]
    wdm_dep_t = dep_W[:, :E_DM].T
    wdm_head_t = head_W[:, :E_DM].T
    dm_emb, dm_dep, dm_head = pl.pallas_call(
        _stage0_kernel,
        out_shape=[
            jax.ShapeDtypeStruct((B * L, E_DM), jnp.float32),
            jax.ShapeDtypeStruct((B * L, REL), jnp.float32),
            jax.ShapeDtypeStruct((B * L, REL), jnp.float32),
        ],
    )(f0, f1, f2, pos_lut, cat_lut, sense_lut, wdm_dep_t, wdm_head_t,
      dep_b.reshape(1, REL), head_b.reshape(1, REL))

    enc = src_enc.reshape(B, L, R, D_SRC)
    idx_col = index.reshape(B, L, 1)
    wsrc_t = dep_W[:, E_DM:].T.astype(jnp.bfloat16)
    whsrc_t = head_W[:, E_DM:].T.astype(jnp.bfloat16)
    dm_dep_b = dm_dep.reshape(B, L, REL)
    dm_head_b = dm_head.reshape(B, L, REL)
    n_a = L // A_TILE
    dep_out, head_out = pl.pallas_call(
        _dep_kernel,
        grid=(B, n_a),
        in_specs=[
            pl.BlockSpec((1, L, 1), lambda b, a: (b, 0, 0)),
            pl.BlockSpec((1, A_TILE, R, D_SRC), lambda b, a: (b, a, 0, 0)),
            pl.BlockSpec((D_SRC, REL), lambda b, a: (0, 0)),
            pl.BlockSpec((D_SRC, REL), lambda b, a: (0, 0)),
            pl.BlockSpec((1, L, REL), lambda b, a: (b, 0, 0)),
            pl.BlockSpec((1, A_TILE, REL), lambda b, a: (b, a, 0)),
        ],
        out_specs=[
            pl.BlockSpec((A_TILE, L, REL), lambda b, a: (b * n_a + a, 0, 0)),
            pl.BlockSpec((A_TILE, REL), lambda b, a: (b * n_a + a, 0)),
        ],
        out_shape=[
            jax.ShapeDtypeStruct((B * L, L, REL), jnp.float32),
            jax.ShapeDtypeStruct((B * L, REL), jnp.float32),
        ],
        compiler_params=pltpu.CompilerParams(
            dimension_semantics=("parallel", "parallel"),
        ),
    )(idx_col, enc, wsrc_t, whsrc_t, dm_dep_b, dm_head_b)
    return (dm_emb, head_out, dep_out)


# speedup vs baseline: 1.0066x; 1.0066x over previous
import jax
import jax.numpy as jnp
from jax.experimental import pallas as pl
from jax.experimental.pallas import tpu as pltpu

B = 4
L = 256
R = 256
D_SRC = 128
E_POS = 64
E_CAT = 64
E_SENSE = 64
E_DM = E_POS + E_CAT + E_SENSE
REL = 256
VOCAB = 50
INP = E_DM + D_SRC

A_TILE = 64


def _main_kernel(idx_smem_ref, idx_ref, enc_ref, wsrc_t_ref, whsrc_t_ref,
                 f0_ref, f1_ref, f2_ref, pos_ref, cat_ref, sense_ref,
                 wdm_dep_t_ref, wdm_head_t_ref, dep_b_ref, head_b_ref,
                 dep_out_ref, head_out_ref, dm_emb_ref,
                 dm_dep_s, dm_head_s):
    b_i = pl.program_id(0)
    a_i = pl.program_id(1)

    @pl.when(jnp.logical_and(b_i == 0, a_i == 0))
    def _stage0():
        n = B * L
        viota = jax.lax.broadcasted_iota(jnp.int32, (n, VOCAB), 1)
        oh0 = (f0_ref[:] == viota).astype(jnp.float32)
        oh1 = (f1_ref[:] == viota).astype(jnp.float32)
        oh2 = (f2_ref[:] == viota).astype(jnp.float32)
        e0 = jnp.dot(oh0, pos_ref[:], preferred_element_type=jnp.float32)
        e1 = jnp.dot(oh1, cat_ref[:], preferred_element_type=jnp.float32)
        e2 = jnp.dot(oh2, sense_ref[:], preferred_element_type=jnp.float32)
        dm = jnp.concatenate([e0, e1, e2], axis=1)
        dm_emb_ref[:, :] = dm
        dm_dep_s[:, :] = jnp.dot(dm, wdm_dep_t_ref[:],
                                 preferred_element_type=jnp.float32) + dep_b_ref[:]
        dm_head_s[:, :] = jnp.dot(dm, wdm_head_t_ref[:],
                                  preferred_element_type=jnp.float32) + head_b_ref[:]

    idxc = idx_ref[0]
    iota = jax.lax.broadcasted_iota(jnp.int32, (L, R), 1)
    onehot = (idxc == iota).astype(jnp.bfloat16)
    wsrc = wsrc_t_ref[:]
    whsrc = whsrc_t_ref[:]
    dmdep = dm_dep_s[pl.ds(b_i * L, L), :]
    rows = []
    for t in range(A_TILE):
        e = enc_ref[0, t].astype(jnp.bfloat16)
        g = jnp.dot(onehot, e, preferred_element_type=jnp.float32)
        dep_out_ref[t] = (
            jnp.dot(g.astype(jnp.bfloat16), wsrc, preferred_element_type=jnp.float32)
            + dmdep
        )
        ag = a_i * A_TILE + t
        iv = idx_smem_ref[b_i, ag]
        rows.append(enc_ref[0, t, pl.ds(iv, 1), :])
    hs = jnp.concatenate(rows, axis=0).astype(jnp.bfloat16)
    dmhead = dm_head_s[pl.ds(b_i * L + a_i * A_TILE, A_TILE), :]
    head_out_ref[:, :] = (
        jnp.dot(hs, whsrc, preferred_element_type=jnp.float32) + dmhead
    )


def kernel(feats, index, src_enc, pos_lut, cat_lut, sense_lut, head_W, head_b, dep_W, dep_b):
    f0 = feats[:, 0:1]
    f1 = feats[:, 1:2]
    f2 = feats[:, 2:3]
    wdm_dep_t = dep_W[:, :E_DM].T
    wdm_head_t = head_W[:, :E_DM].T
    wsrc_t = dep_W[:, E_DM:].T.astype(jnp.bfloat16)
    whsrc_t = head_W[:, E_DM:].T.astype(jnp.bfloat16)
    enc = src_enc.reshape(B, L, R, D_SRC)
    idx_col = index.reshape(B, L, 1)
    n_a = L // A_TILE

    full = lambda b, a: (0, 0)
    dep_out, head_out, dm_emb = pl.pallas_call(
        _main_kernel,
        grid=(B, n_a),
        in_specs=[
            pl.BlockSpec(memory_space=pltpu.SMEM),
            pl.BlockSpec((1, L, 1), lambda b, a: (b, 0, 0)),
            pl.BlockSpec((1, A_TILE, R, D_SRC), lambda b, a: (b, a, 0, 0)),
            pl.BlockSpec((D_SRC, REL), full),
            pl.BlockSpec((D_SRC, REL), full),
            pl.BlockSpec((B * L, 1), full),
            pl.BlockSpec((B * L, 1), full),
            pl.BlockSpec((B * L, 1), full),
            pl.BlockSpec((VOCAB, E_POS), full),
            pl.BlockSpec((VOCAB, E_CAT), full),
            pl.BlockSpec((VOCAB, E_SENSE), full),
            pl.BlockSpec((E_DM, REL), full),
            pl.BlockSpec((E_DM, REL), full),
            pl.BlockSpec((1, REL), full),
            pl.BlockSpec((1, REL), full),
        ],
        out_specs=[
            pl.BlockSpec((A_TILE, L, REL), lambda b, a: (b * n_a + a, 0, 0)),
            pl.BlockSpec((A_TILE, REL), lambda b, a: (b * n_a + a, 0)),
            pl.BlockSpec((B * L, E_DM), full),
        ],
        out_shape=[
            jax.ShapeDtypeStruct((B * L, L, REL), jnp.float32),
            jax.ShapeDtypeStruct((B * L, REL), jnp.float32),
            jax.ShapeDtypeStruct((B * L, E_DM), jnp.float32),
        ],
        scratch_shapes=[
            pltpu.VMEM((B * L, REL), jnp.float32),
            pltpu.VMEM((B * L, REL), jnp.float32),
        ],
        compiler_params=pltpu.CompilerParams(
            dimension_semantics=("arbitrary", "arbitrary"),
        ),
    )(index.reshape(B, L), idx_col, enc, wsrc_t, whsrc_t, f0, f1, f2, pos_lut, cat_lut, sense_lut,
      wdm_dep_t, wdm_head_t, dep_b.reshape(1, REL), head_b.reshape(1, REL))
    return (dm_emb, head_out, dep_out)


# speedup vs baseline: 1.0081x; 1.0015x over previous
import jax
import jax.numpy as jnp
from jax.experimental import pallas as pl
from jax.experimental.pallas import tpu as pltpu

B = 4
L = 256
R = 256
D_SRC = 128
E_POS = 64
E_CAT = 64
E_SENSE = 64
E_DM = E_POS + E_CAT + E_SENSE
REL = 256
VOCAB = 50
INP = E_DM + D_SRC

A_TILE = 64


def _main_kernel(idx_smem_ref, idx_ref, enc_ref, wsrc_t_ref, whsrc_t_ref,
                 f0_ref, f1_ref, f2_ref, pos_ref, cat_ref, sense_ref,
                 wdm_dep_t_ref, wdm_head_t_ref, dep_b_ref, head_b_ref,
                 dep_out_ref, head_out_ref, dm_emb_ref,
                 dm_dep_s, dm_head_s):
    b_i = pl.program_id(0)
    a_i = pl.program_id(1)

    @pl.when(jnp.logical_and(b_i == 0, a_i == 0))
    def _stage0():
        n = B * L
        viota = jax.lax.broadcasted_iota(jnp.int32, (n, VOCAB), 1)
        oh0 = (f0_ref[:] == viota).astype(jnp.float32)
        oh1 = (f1_ref[:] == viota).astype(jnp.float32)
        oh2 = (f2_ref[:] == viota).astype(jnp.float32)
        e0 = jnp.dot(oh0, pos_ref[:], preferred_element_type=jnp.float32)
        e1 = jnp.dot(oh1, cat_ref[:], preferred_element_type=jnp.float32)
        e2 = jnp.dot(oh2, sense_ref[:], preferred_element_type=jnp.float32)
        dm = jnp.concatenate([e0, e1, e2], axis=1)
        dm_emb_ref[:, :] = dm
        dm_dep_s[:, :] = jnp.dot(dm, wdm_dep_t_ref[:],
                                 preferred_element_type=jnp.float32) + dep_b_ref[:]
        dm_head_s[:, :] = jnp.dot(dm, wdm_head_t_ref[:],
                                  preferred_element_type=jnp.float32) + head_b_ref[:]

    idxc = idx_ref[0]
    iota = jax.lax.broadcasted_iota(jnp.int32, (L, R), 1)
    onehot = (idxc == iota).astype(jnp.bfloat16)
    wsrc = wsrc_t_ref[:]
    whsrc = whsrc_t_ref[:]
    dmdep = dm_dep_s[pl.ds(b_i * L, L), :]
    rows = []
    for t in range(A_TILE):
        e = enc_ref[0, t].astype(jnp.bfloat16)
        g = jnp.dot(onehot, e, preferred_element_type=jnp.float32)
        dep_out_ref[t] = (
            jnp.dot(g.astype(jnp.bfloat16), wsrc, preferred_element_type=jnp.float32)
            + dmdep
        )
        ag = a_i * A_TILE + t
        sel = jax.lax.broadcasted_iota(jnp.int32, (L, D_SRC), 0) == ag
        rows.append(jnp.sum(jnp.where(sel, g, 0.0), axis=0, keepdims=True))
    hs = jnp.concatenate(rows, axis=0).astype(jnp.bfloat16)
    dmhead = dm_head_s[pl.ds(b_i * L + a_i * A_TILE, A_TILE), :]
    head_out_ref[:, :] = (
        jnp.dot(hs, whsrc, preferred_element_type=jnp.float32) + dmhead
    )


def kernel(feats, index, src_enc, pos_lut, cat_lut, sense_lut, head_W, head_b, dep_W, dep_b):
    f0 = feats[:, 0:1]
    f1 = feats[:, 1:2]
    f2 = feats[:, 2:3]
    wdm_dep_t = dep_W[:, :E_DM].T
    wdm_head_t = head_W[:, :E_DM].T
    wsrc_t = dep_W[:, E_DM:].T.astype(jnp.bfloat16)
    whsrc_t = head_W[:, E_DM:].T.astype(jnp.bfloat16)
    enc = src_enc.reshape(B, L, R, D_SRC)
    idx_col = index.reshape(B, L, 1)
    n_a = L // A_TILE

    full = lambda b, a: (0, 0)
    dep_out, head_out, dm_emb = pl.pallas_call(
        _main_kernel,
        grid=(B, n_a),
        in_specs=[
            pl.BlockSpec(memory_space=pltpu.SMEM),
            pl.BlockSpec((1, L, 1), lambda b, a: (b, 0, 0)),
            pl.BlockSpec((1, A_TILE, R, D_SRC), lambda b, a: (b, a, 0, 0)),
            pl.BlockSpec((D_SRC, REL), full),
            pl.BlockSpec((D_SRC, REL), full),
            pl.BlockSpec((B * L, 1), full),
            pl.BlockSpec((B * L, 1), full),
            pl.BlockSpec((B * L, 1), full),
            pl.BlockSpec((VOCAB, E_POS), full),
            pl.BlockSpec((VOCAB, E_CAT), full),
            pl.BlockSpec((VOCAB, E_SENSE), full),
            pl.BlockSpec((E_DM, REL), full),
            pl.BlockSpec((E_DM, REL), full),
            pl.BlockSpec((1, REL), full),
            pl.BlockSpec((1, REL), full),
        ],
        out_specs=[
            pl.BlockSpec((A_TILE, L, REL), lambda b, a: (b * n_a + a, 0, 0)),
            pl.BlockSpec((A_TILE, REL), lambda b, a: (b * n_a + a, 0)),
            pl.BlockSpec((B * L, E_DM), full),
        ],
        out_shape=[
            jax.ShapeDtypeStruct((B * L, L, REL), jnp.float32),
            jax.ShapeDtypeStruct((B * L, REL), jnp.float32),
            jax.ShapeDtypeStruct((B * L, E_DM), jnp.float32),
        ],
        scratch_shapes=[
            pltpu.VMEM((B * L, REL), jnp.float32),
            pltpu.VMEM((B * L, REL), jnp.float32),
        ],
        compiler_params=pltpu.CompilerParams(
            dimension_semantics=("arbitrary", "arbitrary"),
            vmem_limit_bytes=100 * 1024 * 1024,
        ),
    )(index.reshape(B, L), idx_col, enc, wsrc_t, whsrc_t, f0, f1, f2, pos_lut, cat_lut, sense_lut,
      wdm_dep_t, wdm_head_t, dep_b.reshape(1, REL), head_b.reshape(1, REL))
    return (dm_emb, head_out, dep_out)


# speedup vs baseline: 1.0175x; 1.0093x over previous
import jax
import jax.numpy as jnp
from jax.experimental import pallas as pl
from jax.experimental.pallas import tpu as pltpu

B = 4
L = 256
R = 256
D_SRC = 128
E_POS = 64
E_CAT = 64
E_SENSE = 64
E_DM = E_POS + E_CAT + E_SENSE
REL = 256
VOCAB = 50
INP = E_DM + D_SRC

A_TILE = 64


def _main_kernel(idx_ref, enc_ref, wsrc_t_ref, whsrc_t_ref,
                 f0_ref, f1_ref, f2_ref, pos_ref, cat_ref, sense_ref,
                 wdm_dep_t_ref, wdm_head_t_ref, dep_b_ref, head_b_ref,
                 dep_out_ref, head_out_ref, dm_emb_ref,
                 dm_dep_s, dm_head_s):
    b_i = pl.program_id(0)
    a_i = pl.program_id(1)

    @pl.when(jnp.logical_and(b_i == 0, a_i == 0))
    def _stage0():
        n = B * L
        viota = jax.lax.broadcasted_iota(jnp.int32, (n, VOCAB), 1)
        oh0 = (f0_ref[:] == viota).astype(jnp.float32)
        oh1 = (f1_ref[:] == viota).astype(jnp.float32)
        oh2 = (f2_ref[:] == viota).astype(jnp.float32)
        e0 = jnp.dot(oh0, pos_ref[:], preferred_element_type=jnp.float32)
        e1 = jnp.dot(oh1, cat_ref[:], preferred_element_type=jnp.float32)
        e2 = jnp.dot(oh2, sense_ref[:], preferred_element_type=jnp.float32)
        dm = jnp.concatenate([e0, e1, e2], axis=1)
        dm_emb_ref[:, :] = dm
        dm_dep_s[:, :] = jnp.dot(dm, wdm_dep_t_ref[:],
                                 preferred_element_type=jnp.float32) + dep_b_ref[:]
        dm_head_s[:, :] = jnp.dot(dm, wdm_head_t_ref[:],
                                  preferred_element_type=jnp.float32) + head_b_ref[:]

    idxc = idx_ref[0]
    iota = jax.lax.broadcasted_iota(jnp.int32, (L, R), 1)
    onehot = (idxc == iota).astype(jnp.bfloat16)
    wsrc = wsrc_t_ref[:]
    whsrc = whsrc_t_ref[:]
    dmdep = dm_dep_s[pl.ds(b_i * L, L), :]
    rows = []
    for t in range(A_TILE):
        e = enc_ref[0, t].astype(jnp.bfloat16)
        g = jnp.dot(onehot, e, preferred_element_type=jnp.float32)
        dep_out_ref[t] = (
            jnp.dot(g.astype(jnp.bfloat16), wsrc, preferred_element_type=jnp.float32)
            + dmdep
        )
        ag = a_i * A_TILE + t
        sel = jax.lax.broadcasted_iota(jnp.int32, (L, D_SRC), 0) == ag
        rows.append(jnp.sum(jnp.where(sel, g, 0.0), axis=0, keepdims=True))
    hs = jnp.concatenate(rows, axis=0).astype(jnp.bfloat16)
    dmhead = dm_head_s[pl.ds(b_i * L + a_i * A_TILE, A_TILE), :]
    head_out_ref[:, :] = (
        jnp.dot(hs, whsrc, preferred_element_type=jnp.float32) + dmhead
    )


def kernel(feats, index, src_enc, pos_lut, cat_lut, sense_lut, head_W, head_b, dep_W, dep_b):
    f0 = feats[:, 0:1]
    f1 = feats[:, 1:2]
    f2 = feats[:, 2:3]
    wdm_dep_t = dep_W[:, :E_DM].T
    wdm_head_t = head_W[:, :E_DM].T
    wsrc_t = dep_W[:, E_DM:].T.astype(jnp.bfloat16)
    whsrc_t = head_W[:, E_DM:].T.astype(jnp.bfloat16)
    enc = src_enc.reshape(B, L, R, D_SRC)
    idx_col = index.reshape(B, L, 1)
    n_a = L // A_TILE

    full = lambda b, a: (0, 0)
    dep_out, head_out, dm_emb = pl.pallas_call(
        _main_kernel,
        grid=(B, n_a),
        in_specs=[
            pl.BlockSpec((1, L, 1), lambda b, a: (b, 0, 0)),
            pl.BlockSpec((1, A_TILE, R, D_SRC), lambda b, a: (b, a, 0, 0)),
            pl.BlockSpec((D_SRC, REL), full),
            pl.BlockSpec((D_SRC, REL), full),
            pl.BlockSpec((B * L, 1), full),
            pl.BlockSpec((B * L, 1), full),
            pl.BlockSpec((B * L, 1), full),
            pl.BlockSpec((VOCAB, E_POS), full),
            pl.BlockSpec((VOCAB, E_CAT), full),
            pl.BlockSpec((VOCAB, E_SENSE), full),
            pl.BlockSpec((E_DM, REL), full),
            pl.BlockSpec((E_DM, REL), full),
            pl.BlockSpec((1, REL), full),
            pl.BlockSpec((1, REL), full),
        ],
        out_specs=[
            pl.BlockSpec((A_TILE, L, REL), lambda b, a: (b * n_a + a, 0, 0)),
            pl.BlockSpec((A_TILE, REL), lambda b, a: (b * n_a + a, 0)),
            pl.BlockSpec((B * L, E_DM), full),
        ],
        out_shape=[
            jax.ShapeDtypeStruct((B * L, L, REL), jnp.float32),
            jax.ShapeDtypeStruct((B * L, REL), jnp.float32),
            jax.ShapeDtypeStruct((B * L, E_DM), jnp.float32),
        ],
        scratch_shapes=[
            pltpu.VMEM((B * L, REL), jnp.float32),
            pltpu.VMEM((B * L, REL), jnp.float32),
        ],
        compiler_params=pltpu.CompilerParams(
            dimension_semantics=("arbitrary", "arbitrary"),
        ),
    )(idx_col, enc, wsrc_t, whsrc_t, f0, f1, f2, pos_lut, cat_lut, sense_lut,
      wdm_dep_t, wdm_head_t, dep_b.reshape(1, REL), head_b.reshape(1, REL))
    return (dm_emb, head_out, dep_out)
